# Initial kernel scaffold; baseline (speedup 1.0000x reference)
import jax
import jax.numpy as jnp
from jax.experimental import pallas as pl


def kernel(predictions, targets):
    raise NotImplementedError("write your pallas kernel here")


# speedup vs baseline: 12.3305x; 12.3305x over previous
import dataclasses
import functools

import jax
import jax.numpy as jnp
from jax import lax
from jax.experimental import pallas as pl
from jax.experimental.pallas import tpu as pltpu
from jax.experimental.pallas import tpu_sc as plsc

_C = 64


def _argmax_body(p_ref, o_ref):
    o_ref[...] = jnp.argmax(p_ref[...], axis=-1).astype(jnp.int32)


def _argmax(predictions, block_b=128):
    B, W, C = predictions.shape
    return pl.pallas_call(
        _argmax_body,
        grid=(B // block_b,),
        in_specs=[pl.BlockSpec((block_b, W, C), lambda i: (i, 0, 0))],
        out_specs=pl.BlockSpec((block_b, W), lambda i: (i, 0)),
        out_shape=jax.ShapeDtypeStruct((B, W), jnp.int32),
    )(predictions)


_NC, _NS, _L = 2, 16, 16
_NW = _NC * _NS


def _sc_loss_call(pred_classes, targets):
    B, W = targets.shape
    rows = B // _NW
    groups = rows // _L
    mesh = plsc.VectorSubcoreMesh(core_axis_name="c", subcore_axis_name="s")
    cp_params = pltpu.CompilerParams()
    if "needs_layout_passes" in pltpu.CompilerParams.__dataclass_fields__:
        cp_params = dataclasses.replace(cp_params, needs_layout_passes=False)

    @functools.partial(
        pl.kernel,
        mesh=mesh,
        compiler_params=cp_params,
        out_type=jax.ShapeDtypeStruct((_NW, _L), jnp.float32),
        scratch_types=[
            pltpu.VMEM((rows, W), jnp.int32),
            pltpu.VMEM((rows, W), jnp.int32),
            pltpu.VMEM((rows, _C), jnp.int32),
            pltpu.VMEM((rows, _C), jnp.int32),
            pltpu.VMEM((_L,), jnp.float32),
            pltpu.SemaphoreType.DMA,
        ],
    )
    def body(pred_hbm, true_hbm, out_hbm, pvm, tvm, ptab, ttab, osc, sem):
        wid = lax.axis_index("s") * _NC + lax.axis_index("c")
        base = wid * rows
        cp = pltpu.async_copy(pred_hbm.at[pl.ds(base, rows)], pvm, sem)
        ct = pltpu.async_copy(true_hbm.at[pl.ds(base, rows)], tvm, sem)

        zeros = jnp.zeros((_L,), jnp.int32)

        @pl.loop(0, rows)
        def _(r):
            for c0 in range(0, _C, _L):
                ptab[r, pl.ds(c0, _L)] = zeros
                ttab[r, pl.ds(c0, _L)] = zeros

        cp.wait()
        ct.wait()

        lanes = lax.iota(jnp.int32, _L)
        rowvecs = [lanes + g * _L for g in range(groups)]
        ones = jnp.ones((_L,), jnp.int32)

        def step(i, st):
            col = jnp.full((_L,), 0, jnp.int32) + i
            new = []
            for t, (vm, tab) in enumerate(((pvm, ptab), (tvm, ttab))):
                per_t = []
                for g in range(groups):
                    prev, run = st[t][g]
                    c = plsc.load_gather(vm, [rowvecs[g], col])
                    run = jnp.where(c == prev, run + ones, ones)
                    cur = plsc.load_gather(tab, [rowvecs[g], c])
                    plsc.store_scatter(tab, [rowvecs[g], c],
                                       jnp.maximum(cur, run))
                    per_t.append((c, run))
                new.append(tuple(per_t))
            return tuple(new)

        neg = jnp.full((_L,), -1, jnp.int32)
        init = tuple(tuple((neg, zeros) for _ in range(groups))
                     for _ in range(2))
        lax.fori_loop(0, W, step, init)

        def loss_r(r, acc):
            for c0 in range(0, _C, _L):
                d = (ptab[r, pl.ds(c0, _L)]
                     - ttab[r, pl.ds(c0, _L)]).astype(jnp.float32)
                acc = acc + d * d
            return acc

        acc = lax.fori_loop(0, rows, loss_r, jnp.zeros((_L,), jnp.float32))
        osc[...] = acc
        pltpu.sync_copy(osc, out_hbm.at[wid])

    return body(pred_classes, targets)


def kernel(predictions, targets):
    B, W, C = predictions.shape
    pred_classes = _argmax(predictions)
    partials = _sc_loss_call(pred_classes, targets)
    return jnp.sum(partials) / (B * C)


# speedup vs baseline: 26.7529x; 2.1697x over previous
import dataclasses
import functools

import jax
import jax.numpy as jnp
from jax import lax
from jax.experimental import pallas as pl
from jax.experimental.pallas import tpu as pltpu
from jax.experimental.pallas import tpu_sc as plsc

_C = 64


def _argmax_body(p_ref, o_ref):
    o_ref[...] = jnp.argmax(p_ref[...], axis=1).astype(jnp.int32)


def _argmax_wcb(pt, block_w=8):
    W, C, B = pt.shape
    return pl.pallas_call(
        _argmax_body,
        grid=(W // block_w,),
        in_specs=[pl.BlockSpec((block_w, C, B), lambda i: (i, 0, 0))],
        out_specs=pl.BlockSpec((block_w, B), lambda i: (i, 0)),
        out_shape=jax.ShapeDtypeStruct((W, B), jnp.int32),
    )(pt)


_NC, _NS, _L = 2, 16, 16
_NW = _NC * _NS


def _sc_loss_call(pred_bm, true_bm):
    B, W = true_bm.shape
    rows = B // _NW
    groups = rows // _L
    mesh = plsc.VectorSubcoreMesh(core_axis_name="c", subcore_axis_name="s")
    cp_params = pltpu.CompilerParams()
    if "needs_layout_passes" in pltpu.CompilerParams.__dataclass_fields__:
        cp_params = dataclasses.replace(cp_params, needs_layout_passes=False)

    @functools.partial(
        pl.kernel,
        mesh=mesh,
        compiler_params=cp_params,
        out_type=jax.ShapeDtypeStruct((_NW, _L), jnp.float32),
        scratch_types=[
            pltpu.VMEM((rows, W), jnp.int32),
            pltpu.VMEM((rows, W), jnp.int32),
            pltpu.VMEM((rows, _C), jnp.int32),
            pltpu.VMEM((rows, _C), jnp.int32),
            pltpu.VMEM((_L,), jnp.float32),
            pltpu.SemaphoreType.DMA,
        ],
    )
    def body(pred_hbm, true_hbm, out_hbm, pvm, tvm, ptab, ttab, osc, sem):
        wid = lax.axis_index("s") * _NC + lax.axis_index("c")
        base = wid * rows
        cp = pltpu.async_copy(pred_hbm.at[pl.ds(base, rows)], pvm, sem)
        ct = pltpu.async_copy(true_hbm.at[pl.ds(base, rows)], tvm, sem)

        zeros = jnp.zeros((_L,), jnp.int32)

        @pl.loop(0, rows)
        def _(r):
            for c0 in range(0, _C, _L):
                ptab[r, pl.ds(c0, _L)] = zeros
                ttab[r, pl.ds(c0, _L)] = zeros

        cp.wait()
        ct.wait()

        lanes = lax.iota(jnp.int32, _L)
        rowvecs = [lanes + g * _L for g in range(groups)]
        ones = jnp.ones((_L,), jnp.int32)

        def step(i, st):
            col = jnp.full((_L,), 0, jnp.int32) + i
            new = []
            for t, (vm, tab) in enumerate(((pvm, ptab), (tvm, ttab))):
                per_t = []
                for g in range(groups):
                    prev, run = st[t][g]
                    c = plsc.load_gather(vm, [rowvecs[g], col])
                    run = jnp.where(c == prev, run + ones, ones)
                    cur = plsc.load_gather(tab, [rowvecs[g], c])
                    plsc.store_scatter(tab, [rowvecs[g], c],
                                       jnp.maximum(cur, run))
                    per_t.append((c, run))
                new.append(tuple(per_t))
            return tuple(new)

        neg = jnp.full((_L,), -1, jnp.int32)
        init = tuple(tuple((neg, zeros) for _ in range(groups))
                     for _ in range(2))
        lax.fori_loop(0, W, step, init)

        def loss_r(r, acc):
            for c0 in range(0, _C, _L):
                d = (ptab[r, pl.ds(c0, _L)]
                     - ttab[r, pl.ds(c0, _L)]).astype(jnp.float32)
                acc = acc + d * d
            return acc

        acc = lax.fori_loop(0, rows, loss_r, jnp.zeros((_L,), jnp.float32))
        osc[...] = acc
        pltpu.sync_copy(osc, out_hbm.at[wid])

    return body(pred_bm, true_bm)


def kernel(predictions, targets):
    B, W, C = predictions.shape
    pt = jnp.transpose(predictions, (1, 2, 0))
    pred_wm = _argmax_wcb(pt)
    partials = _sc_loss_call(pred_wm.T, targets)
    return jnp.sum(partials) / (B * C)
